# Initial kernel scaffold; baseline (speedup 1.0000x reference)
import jax
import jax.numpy as jnp
from jax.experimental import pallas as pl


def kernel(ent_embeds, rel_embeds, global_emb_list, w_bases1, w_comp1, loop_w1, w_bases2, w_comp2, loop_w2, edge_index, edge_type, node_ids_graph, s_len, s_tem, r_tem, reverse):
    raise NotImplementedError("write your pallas kernel here")


# speedup vs baseline: 2.1031x; 2.1031x over previous
import jax
import jax.numpy as jnp
from jax.experimental import pallas as pl
from jax.experimental.pallas import tpu as pltpu

H = 128
NB = 16
NREL = 230
N = 10000
E = 40000
B = 2048
SEQ = 10
TT = B * SEQ


def _matmul_kernel(x_ref, w_ref, o_ref):
    o_ref[...] = jnp.dot(x_ref[...], w_ref[...],
                         preferred_element_type=jnp.float32)


def _mm(x, w, block_m):
    m, k = x.shape
    k2, n = w.shape
    grid = (m // block_m,)
    return pl.pallas_call(
        _matmul_kernel,
        grid=grid,
        in_specs=[
            pl.BlockSpec((block_m, k), lambda i: (i, 0)),
            pl.BlockSpec((k, n), lambda i: (0, 0)),
        ],
        out_specs=pl.BlockSpec((block_m, n), lambda i: (i, 0)),
        out_shape=jax.ShapeDtypeStruct((m, n), jnp.float32),
    )(x, w)


def _rgcn_layer(x, src, dst, etype, w_bases, w_comp, loop_w, activation):
    w_all = jnp.transpose(w_bases, (1, 0, 2)).reshape(H, NB * H)
    xb = _mm(x, w_all, 1000).reshape(N, NB, H)
    coef = jnp.take(w_comp, etype, axis=0)
    msg = jnp.einsum('eb,ebd->ed', coef, jnp.take(xb, src, axis=0))
    agg = jax.ops.segment_sum(msg, dst, num_segments=N)
    deg = jax.ops.segment_sum(jnp.ones((E,), jnp.float32), dst, num_segments=N)
    agg = agg / jnp.clip(deg, 1.0, None)[:, None]
    h = agg + _mm(x, loop_w, 1000)
    if activation:
        h = jax.nn.relu(h)
    return h


def kernel(ent_embeds, rel_embeds, global_emb_list, w_bases1, w_comp1, loop_w1,
           w_bases2, w_comp2, loop_w2, edge_index, edge_type, node_ids_graph,
           s_len, s_tem, r_tem, reverse):
    src = edge_index[0]
    dst = edge_index[1]
    et = edge_type + reverse * NREL
    h = _rgcn_layer(ent_embeds, src, dst, et, w_bases1, w_comp1, loop_w1, True)
    h = _rgcn_layer(h, src, dst, et, w_bases2, w_comp2, loop_w2, False)
    embeds_tok = jnp.take(h, node_ids_graph, axis=0)
    b_idx = jnp.arange(TT, dtype=jnp.int32) // SEQ
    ent_s = jnp.take(jnp.take(ent_embeds, s_tem, axis=0), b_idx, axis=0)
    rel_r = jnp.take(jnp.take(rel_embeds, r_tem, axis=0), b_idx, axis=0)
    feat = jnp.concatenate([embeds_tok, ent_s, rel_r, global_emb_list], axis=1)
    feat_r = jnp.concatenate([embeds_tok, ent_s, global_emb_list], axis=1)
    t = feat.reshape(B, SEQ, 4 * H)
    t_r = feat_r.reshape(B, SEQ, 3 * H)
    return (t, t_r)
